# baseline (speedup 1.0000x reference)
import functools

import jax
import jax.numpy as jnp
from jax import lax
from jax.experimental import pallas as pl
from jax.experimental.pallas import tpu as pltpu
from jax.experimental.pallas import tpu_sc as plsc

N = 10000
E = 320000
D = 128

NC = 2
NS = 16
NW = NC * NS
K = 128
CH = 80
E_PAD = NW * CH * K
NBUF = 2
HC = CH // 2
DUMMY = N
NPAD = 10240
SLAB = NPAD // NS
_BM = 256


@functools.cache
def _build():
    mesh = plsc.VectorSubcoreMesh(core_axis_name="c", subcore_axis_name="s",
                                  num_cores=NC, num_subcores=NS)

    @functools.partial(
        pl.kernel,
        out_type=jax.ShapeDtypeStruct((2, NW, NPAD), jnp.float32),
        mesh=mesh,
        compiler_params=pltpu.CompilerParams(needs_layout_passes=False),
        scratch_types=[
            pltpu.VMEM((CH, K), jnp.int32),
            pltpu.VMEM((NPAD,), jnp.float32),
        ],
    )
    def degrees(e3, out, idx_v, hist_v):
        c = lax.axis_index("c")
        s = lax.axis_index("s")
        wid = c * NS + s
        zeros16 = jnp.zeros((16,), jnp.float32)
        for kind in range(2):
            def zbody(j, carry):
                hist_v[pl.ds(j * 16, 16)] = zeros16
                return carry

            lax.fori_loop(0, NPAD // 16, zbody, 0)
            pltpu.sync_copy(e3.at[kind].at[wid], idx_v)

            def cbody(j, carry):
                for i in range(K // 16):
                    idx16 = idx_v[j, pl.ds(i * 16, 16)]
                    counts, last = plsc.scan_count(idx16)
                    plsc.addupdate_scatter(hist_v, [idx16],
                                           counts.astype(jnp.float32),
                                           mask=last)
                return carry

            lax.fori_loop(0, CH, cbody, 0)
            pltpu.sync_copy(hist_v, out.at[kind].at[wid])

    def dense_body(x_ref, w_ref, b_ref, dp_ref, scaled_ref):
        nodes = jnp.dot(x_ref[...], w_ref[...],
                        preferred_element_type=jnp.float32) + b_ref[...][None, :]
        deg_s = jnp.sum(dp_ref[0], axis=0)
        inv_s = lax.rsqrt(jnp.maximum(deg_s, 1.0))
        scaled_ref[...] = nodes * inv_s[:, None]

    dense = pl.pallas_call(
        dense_body,
        grid=(NPAD // _BM,),
        in_specs=[
            pl.BlockSpec((_BM, D), lambda i: (i, 0)),
            pl.BlockSpec((D, D), lambda i: (0, 0)),
            pl.BlockSpec((D,), lambda i: (0,)),
            pl.BlockSpec((2, NW, _BM), lambda i: (0, 0, i)),
        ],
        out_specs=pl.BlockSpec((_BM, D), lambda i: (i, 0)),
        out_shape=jax.ShapeDtypeStruct((NPAD, D), jnp.float32),
    )

    @functools.partial(
        pl.kernel,
        out_type=jax.ShapeDtypeStruct((NC, NPAD, D), jnp.float32),
        mesh=mesh,
        scratch_types=[
            pltpu.VMEM((HC, K), jnp.int32),
            pltpu.VMEM((HC, K), jnp.int32),
            pltpu.VMEM((K, D), jnp.float32),
            pltpu.VMEM((K, D), jnp.float32),
            pltpu.VMEM_SHARED((NPAD, D), jnp.float32),
            pltpu.SemaphoreType.DMA,
            pltpu.SemaphoreType.DMA,
        ],
    )
    def aggregate(e3, scaled, zeros_rows, out, snd_v, rcv_v,
                  r0, r1, acc, s0, s1):
        rows_v = (r0, r1)
        sems = (s0, s1)
        c = lax.axis_index("c")
        s = lax.axis_index("s")
        wid = c * NS + s
        pltpu.sync_copy(zeros_rows, acc.at[pl.ds(s * SLAB, SLAB)])
        plsc.subcore_barrier()
        for h in range(2):
            pltpu.sync_copy(e3.at[0].at[wid].at[pl.ds(h * HC, HC)], snd_v)
            pltpu.sync_copy(e3.at[1].at[wid].at[pl.ds(h * HC, HC)], rcv_v)
            for b in range(NBUF):
                pltpu.async_copy(scaled.at[snd_v.at[b]], rows_v[b], sems[b])

            def body(g, carry):
                for b in range(NBUF):
                    j = g * NBUF + b
                    pltpu.make_async_copy(scaled.at[snd_v.at[j]], rows_v[b],
                                          sems[b]).wait()
                    nj = j + NBUF

                    @pl.when(nj < HC)
                    def _():
                        pltpu.async_copy(scaled.at[snd_v.at[nj]], rows_v[b],
                                         sems[b])

                    pltpu.sync_copy(rows_v[b], acc.at[rcv_v.at[j]], add=True)
                return carry

            lax.fori_loop(0, HC // NBUF, body, 0)
        plsc.subcore_barrier()
        pltpu.sync_copy(acc.at[pl.ds(s * SLAB, SLAB)],
                        out.at[c].at[pl.ds(s * SLAB, SLAB)])

    def final_body(p_ref, dp_ref, o_ref):
        deg_r = jnp.sum(dp_ref[1], axis=0)
        inv_r = lax.rsqrt(jnp.maximum(deg_r, 1.0))
        o_ref[...] = (p_ref[0] + p_ref[1]) * inv_r[:, None]

    final = pl.pallas_call(
        final_body,
        grid=(NPAD // _BM,),
        in_specs=[
            pl.BlockSpec((NC, _BM, D), lambda i: (0, i, 0)),
            pl.BlockSpec((2, NW, _BM), lambda i: (0, 0, i)),
        ],
        out_specs=pl.BlockSpec((_BM, D), lambda i: (i, 0)),
        out_shape=jax.ShapeDtypeStruct((N, D), jnp.float32),
    )

    return degrees, dense, aggregate, final


def kernel(x, edge_index, W, b):
    degrees, dense, aggregate, final = _build()
    e3 = jnp.pad(edge_index, ((0, 0), (0, E_PAD - E)),
                 constant_values=DUMMY).reshape(2, NW, CH, K)
    zeros_rows = jnp.zeros((SLAB, D), jnp.float32)
    dp = degrees(e3)
    scaled = dense(x, W, b, dp)
    parts = aggregate(e3, scaled, zeros_rows)
    return final(parts, dp)


# speedup vs baseline: 1.3526x; 1.3526x over previous
import functools

import jax
import jax.numpy as jnp
from jax import lax
from jax.experimental import pallas as pl
from jax.experimental.pallas import tpu as pltpu
from jax.experimental.pallas import tpu_sc as plsc

N = 10000
E = 320000
D = 128

NC = 2
NS = 16
NW = NC * NS
K = 128
CH = (E + NW * K - 1) // (NW * K)
E_PAD = NW * CH * K
DUMMY = N
NPAD = 10240
SLAB = NPAD // NS
_BM = 256


@functools.cache
def _build():
    mesh = plsc.VectorSubcoreMesh(core_axis_name="c", subcore_axis_name="s",
                                  num_cores=NC, num_subcores=NS)

    @functools.partial(
        pl.kernel,
        out_type=jax.ShapeDtypeStruct((2, NW, NPAD), jnp.float32),
        mesh=mesh,
        compiler_params=pltpu.CompilerParams(needs_layout_passes=False),
        scratch_types=[
            pltpu.VMEM((CH, K), jnp.int32),
            pltpu.VMEM((NPAD,), jnp.float32),
        ],
    )
    def degrees(e3, out, idx_v, hist_v):
        c = lax.axis_index("c")
        s = lax.axis_index("s")
        wid = c * NS + s
        zeros16 = jnp.zeros((16,), jnp.float32)
        for kind in range(2):
            def zbody(j, carry):
                hist_v[pl.ds(j * 16, 16)] = zeros16
                return carry

            lax.fori_loop(0, NPAD // 16, zbody, 0)
            pltpu.sync_copy(e3.at[kind].at[wid], idx_v)

            def cbody(j, carry):
                for i in range(K // 16):
                    idx16 = idx_v[j, pl.ds(i * 16, 16)]
                    counts, last = plsc.scan_count(idx16)
                    plsc.addupdate_scatter(hist_v, [idx16],
                                           counts.astype(jnp.float32),
                                           mask=last)
                return carry

            lax.fori_loop(0, CH, cbody, 0)
            pltpu.sync_copy(hist_v, out.at[kind].at[wid])

    def dense_body(x_ref, w_ref, b_ref, dp_ref, scaled_ref):
        nodes = jnp.dot(x_ref[...], w_ref[...],
                        preferred_element_type=jnp.float32) + b_ref[...][None, :]
        deg_s = jnp.sum(dp_ref[0], axis=0)
        inv_s = lax.rsqrt(jnp.maximum(deg_s, 1.0))
        scaled_ref[...] = nodes * inv_s[:, None]

    dense = pl.pallas_call(
        dense_body,
        grid=(NPAD // _BM,),
        in_specs=[
            pl.BlockSpec((_BM, D), lambda i: (i, 0)),
            pl.BlockSpec((D, D), lambda i: (0, 0)),
            pl.BlockSpec((D,), lambda i: (0,)),
            pl.BlockSpec((2, NW, _BM), lambda i: (0, 0, i)),
        ],
        out_specs=pl.BlockSpec((_BM, D), lambda i: (i, 0)),
        out_shape=jax.ShapeDtypeStruct((NPAD, D), jnp.float32),
    )

    @functools.partial(
        pl.kernel,
        out_type=jax.ShapeDtypeStruct((NC, NPAD, D), jnp.float32),
        mesh=mesh,
        scratch_types=[
            pltpu.VMEM((CH, K), jnp.int32),
            pltpu.VMEM((CH, K), jnp.int32),
            pltpu.VMEM((K, D), jnp.float32),
            pltpu.VMEM_SHARED((NPAD, D), jnp.float32),
            pltpu.SemaphoreType.DMA,
        ],
    )
    def aggregate(e3, scaled, zeros_rows, out, snd_v, rcv_v, rows_v, acc, sem):
        c = lax.axis_index("c")
        s = lax.axis_index("s")
        wid = c * NS + s
        pltpu.sync_copy(zeros_rows, acc.at[pl.ds(s * SLAB, SLAB)])
        plsc.subcore_barrier()
        pltpu.sync_copy(e3.at[0].at[wid], snd_v)
        pltpu.sync_copy(e3.at[1].at[wid], rcv_v)

        def body(j, carry):
            pltpu.async_copy(scaled.at[snd_v.at[j]], rows_v, sem).wait()
            pltpu.sync_copy(rows_v, acc.at[rcv_v.at[j]], add=True)
            return carry

        lax.fori_loop(0, CH, body, 0)
        plsc.subcore_barrier()
        pltpu.sync_copy(acc.at[pl.ds(s * SLAB, SLAB)],
                        out.at[c].at[pl.ds(s * SLAB, SLAB)])

    def final_body(p_ref, dp_ref, o_ref):
        deg_r = jnp.sum(dp_ref[1], axis=0)
        inv_r = lax.rsqrt(jnp.maximum(deg_r, 1.0))
        o_ref[...] = (p_ref[0] + p_ref[1]) * inv_r[:, None]

    final = pl.pallas_call(
        final_body,
        grid=(NPAD // _BM,),
        in_specs=[
            pl.BlockSpec((NC, _BM, D), lambda i: (0, i, 0)),
            pl.BlockSpec((2, NW, _BM), lambda i: (0, 0, i)),
        ],
        out_specs=pl.BlockSpec((_BM, D), lambda i: (i, 0)),
        out_shape=jax.ShapeDtypeStruct((N, D), jnp.float32),
    )

    return degrees, dense, aggregate, final


def kernel(x, edge_index, W, b):
    degrees, dense, aggregate, final = _build()
    e3 = jnp.pad(edge_index, ((0, 0), (0, E_PAD - E)),
                 constant_values=DUMMY).reshape(2, NW, CH, K)
    zeros_rows = jnp.zeros((SLAB, D), jnp.float32)
    dp = degrees(e3)
    scaled = dense(x, W, b, dp)
    parts = aggregate(e3, scaled, zeros_rows)
    return final(parts, dp)
